# baseline (speedup 1.0000x reference)
import jax
import jax.numpy as jnp
from jax.experimental import pallas as pl
from jax.experimental.pallas import tpu as pltpu

B_ORT, B_REG, B_CLS = 0.2, 2.0, 1.0
_B = 4
_P = 40000
_NS = 4
_PC = _P // _NS
_SUB = 8
_LANE = _PC // _SUB
_CLS_TOTAL = float(_B * 2 * _P)


def _partial_kernel(x_ref, t_ref, rg_ref, rt_ref, out_ref, acc_ref):
    b = pl.program_id(0)

    @pl.when(b == 0)
    def _init():
        for i in range(4):
            acc_ref[i] = 0.0

    x = x_ref[0]
    t = t_ref[0]
    p = jax.nn.sigmoid(x)
    pt = jnp.where(t == 1.0, p, 1.0 - p)
    at = jnp.where(t == 1.0, 1000.0, 1.0)
    q = 1.0 - pt
    w = at * q * q
    bce = jnp.maximum(x, 0.0) - x * t + jnp.log1p(jnp.exp(-jnp.abs(x)))
    acc_ref[0] += jnp.sum(w * bce)

    sl1_sum = 0.0
    ce_sum = 0.0
    npos = 0.0
    for a in range(2):
        mask = (rt_ref[0, 9 * a] == 1.0).astype(jnp.float32)
        npos += jnp.sum(mask)
        for j in range(7):
            s = rg_ref[0, 9 * a + j]
            if a == 0 and j == 6:
                s = jnp.tanh(s)
            d = s - rt_ref[0, 9 * a + j + 1]
            ad = jnp.abs(d)
            sl1 = jnp.where(ad < 1.0, 0.5 * d * d, ad - 0.5)
            sl1_sum += jnp.sum(sl1 * mask)
        z0 = rg_ref[0, 9 * a + 7]
        z1 = rg_ref[0, 9 * a + 8]
        tc = rt_ref[0, 9 * a + 8]
        diff = jnp.where(tc == 1.0, z0 - z1, z1 - z0)
        ce = jnp.maximum(diff, 0.0) + jnp.log1p(jnp.exp(-jnp.abs(diff)))
        ce_sum += jnp.sum(ce * mask)

    acc_ref[1] += sl1_sum
    acc_ref[2] += ce_sum
    acc_ref[3] += npos

    @pl.when(b == _B - 1)
    def _final():
        out_ref[...] = jnp.stack(
            [acc_ref[0], acc_ref[1], acc_ref[2], acc_ref[3]]
        ).reshape(1, 4)


def _partial_call(x, t, rg, rt):
    return pl.pallas_call(
        _partial_kernel,
        grid=(_B,),
        in_specs=[
            pl.BlockSpec((1, 2, _SUB, _LANE), lambda b: (b, 0, 0, 0)),
            pl.BlockSpec((1, 2, _SUB, _LANE), lambda b: (b, 0, 0, 0)),
            pl.BlockSpec((1, 18, _SUB, _LANE), lambda b: (b, 0, 0, 0)),
            pl.BlockSpec((1, 18, _SUB, _LANE), lambda b: (b, 0, 0, 0)),
        ],
        out_specs=pl.BlockSpec((1, 4), lambda b: (0, 0)),
        out_shape=jax.ShapeDtypeStruct((1, 4), jnp.float32),
        scratch_shapes=[pltpu.SMEM((4,), jnp.float32)],
    )(x, t, rg, rt)


def _combine_kernel(p_ref, out_ref):
    cls_sum = jnp.sum(p_ref[:, 0])
    sl1_sum = jnp.sum(p_ref[:, 1])
    ce_sum = jnp.sum(p_ref[:, 2])
    n_pos = jnp.sum(p_ref[:, 3])
    cls_loss = cls_sum / _CLS_TOTAL
    reg_loss = sl1_sum / (n_pos * 7.0)
    ort_loss = ce_sum / n_pos
    loss = B_CLS * cls_loss + B_ORT * ort_loss + B_REG * reg_loss
    out_ref[...] = jnp.full((1, 1), loss, dtype=jnp.float32)


def kernel(cls_tensor, reg_tensor, cls_targets, reg_targets):
    xf = cls_tensor.reshape(_B, 2, _P)
    tf = cls_targets.reshape(_B, _P, 2)
    rgf = reg_tensor.reshape(_B, 18, _P)
    rtf = reg_targets.reshape(_B, _P, 2, 9)

    partials = []
    for s in range(_NS):
        sl = slice(s * _PC, (s + 1) * _PC)
        x = xf[:, :, sl].reshape(_B, 2, _SUB, _LANE)
        t = tf[:, sl].transpose(0, 2, 1).reshape(_B, 2, _SUB, _LANE)
        rg = rgf[:, :, sl].reshape(_B, 18, _SUB, _LANE)
        rt = (rtf[:, sl].transpose(0, 2, 3, 1).reshape(_B, 18, _SUB, _LANE))
        partials.append(_partial_call(x, t, rg, rt))

    pstack = jnp.concatenate(partials, axis=0)
    out = pl.pallas_call(
        _combine_kernel,
        out_shape=jax.ShapeDtypeStruct((1, 1), jnp.float32),
    )(pstack)
    return out[0, 0]


# speedup vs baseline: 1.1160x; 1.1160x over previous
import jax
import jax.numpy as jnp
import numpy as np
from jax.experimental import pallas as pl
from jax.experimental.pallas import tpu as pltpu

B_ORT, B_REG, B_CLS = 0.2, 2.0, 1.0
_B = 4
_P = 40000
_R = 625
_Q = 64
_CLS_TOTAL = float(_B * 2 * _P)


def _build_selectors():
    s_rt = np.zeros((1152, 1152), np.float32)
    for q in range(128):
        for c in range(9):
            k = 9 * (q % 2) + c
            s_rt[9 * q + c, 64 * k + q // 2] = 1.0
    s_t = np.zeros((128, 128), np.float32)
    for l in range(128):
        s_t[l, 64 * (l % 2) + l // 2] = 1.0
    return (jnp.asarray(s_rt, dtype=jnp.bfloat16),
            jnp.asarray(s_t, dtype=jnp.bfloat16))


def _loss_kernel(x_ref, t_ref, rg_ref, rt_ref, srt_ref, st_ref,
                 out_ref, acc_ref):
    b = pl.program_id(0)

    @pl.when(b == 0)
    def _init():
        for i in range(4):
            acc_ref[i] = 0.0

    d_rt = jax.lax.dot_general(
        rt_ref[0].astype(jnp.bfloat16), srt_ref[...],
        (((1,), (0,)), ((), ())), preferred_element_type=jnp.float32)
    d_t = jax.lax.dot_general(
        t_ref[0].astype(jnp.bfloat16), st_ref[...],
        (((1,), (0,)), ((), ())), preferred_element_type=jnp.float32)

    def slab(arr, k):
        return jax.lax.slice(arr, (0, _Q * k), (_R, _Q * (k + 1)))

    cls_sum = 0.0
    for c in range(2):
        x = x_ref[0, c]
        t = slab(d_t, c)
        p = jax.nn.sigmoid(x)
        pt = jnp.where(t == 1.0, p, 1.0 - p)
        at = jnp.where(t == 1.0, 1000.0, 1.0)
        qf = 1.0 - pt
        w = at * qf * qf
        bce = jnp.maximum(x, 0.0) - x * t + jnp.log1p(jnp.exp(-jnp.abs(x)))
        cls_sum += jnp.sum(w * bce)

    sl1_sum = 0.0
    ce_sum = 0.0
    npos = 0.0
    for a in range(2):
        mask = (slab(d_rt, 9 * a) == 1.0).astype(jnp.float32)
        npos += jnp.sum(mask)
        for j in range(7):
            s = rg_ref[0, 9 * a + j]
            if a == 0 and j == 6:
                s = jnp.tanh(s)
            d = s - slab(d_rt, 9 * a + j + 1)
            ad = jnp.abs(d)
            sl1 = jnp.where(ad < 1.0, 0.5 * d * d, ad - 0.5)
            sl1_sum += jnp.sum(sl1 * mask)
        z0 = rg_ref[0, 9 * a + 7]
        z1 = rg_ref[0, 9 * a + 8]
        tc = slab(d_rt, 9 * a + 8)
        diff = jnp.where(tc == 1.0, z0 - z1, z1 - z0)
        ce = jnp.maximum(diff, 0.0) + jnp.log1p(jnp.exp(-jnp.abs(diff)))
        ce_sum += jnp.sum(ce * mask)

    acc_ref[0] += cls_sum
    acc_ref[1] += sl1_sum
    acc_ref[2] += ce_sum
    acc_ref[3] += npos

    @pl.when(b == _B - 1)
    def _final():
        n_pos = acc_ref[3]
        cls_loss = acc_ref[0] / _CLS_TOTAL
        reg_loss = acc_ref[1] / (n_pos * 7.0)
        ort_loss = acc_ref[2] / n_pos
        loss = B_CLS * cls_loss + B_ORT * ort_loss + B_REG * reg_loss
        out_ref[...] = jnp.full((1, 1), loss, dtype=jnp.float32)


def kernel(cls_tensor, reg_tensor, cls_targets, reg_targets):
    x = cls_tensor.reshape(_B, 2, _R, _Q)
    t = cls_targets.reshape(_B, _R, 128)
    rg = reg_tensor.reshape(_B, 18, _R, _Q)
    rt = reg_targets.reshape(_B, _R, 1152)
    s_rt, s_t = _build_selectors()

    out = pl.pallas_call(
        _loss_kernel,
        grid=(_B,),
        in_specs=[
            pl.BlockSpec((1, 2, _R, _Q), lambda b: (b, 0, 0, 0)),
            pl.BlockSpec((1, _R, 128), lambda b: (b, 0, 0)),
            pl.BlockSpec((1, 18, _R, _Q), lambda b: (b, 0, 0, 0)),
            pl.BlockSpec((1, _R, 1152), lambda b: (b, 0, 0)),
            pl.BlockSpec((1152, 1152), lambda b: (0, 0)),
            pl.BlockSpec((128, 128), lambda b: (0, 0)),
        ],
        out_specs=pl.BlockSpec((1, 1), lambda b: (0, 0)),
        out_shape=jax.ShapeDtypeStruct((1, 1), jnp.float32),
        scratch_shapes=[pltpu.SMEM((4,), jnp.float32)],
    )(x, t, rg, rt, s_rt, s_t)
    return out[0, 0]


# speedup vs baseline: 2.7371x; 2.4525x over previous
import jax
import jax.numpy as jnp
import numpy as np
from jax.experimental import pallas as pl
from jax.experimental.pallas import tpu as pltpu

B_ORT, B_REG, B_CLS = 0.2, 2.0, 1.0
_B = 4
_P = 40000
_SUB, _LANE = 8, 5000
_R, _Q = 625, 64
_CLS_TOTAL = float(_B * 2 * _P)


def _build_selector():
    s_t = np.zeros((128, 128), np.float32)
    for l in range(128):
        s_t[l, 64 * (l % 2) + l // 2] = 1.0
    return jnp.asarray(s_t, dtype=jnp.bfloat16)


def _loss_kernel(x_ref, t_ref, rg_ref, rt_ref, st_ref, out_ref, acc_ref):
    b = pl.program_id(0)

    @pl.when(b == 0)
    def _init():
        for i in range(4):
            acc_ref[i] = 0.0

    d_t = jax.lax.dot_general(
        t_ref[0].astype(jnp.bfloat16), st_ref[...],
        (((1,), (0,)), ((), ())), preferred_element_type=jnp.float32)
    cls_sum = 0.0
    for c in range(2):
        x = x_ref[0, c]
        t = jax.lax.slice(d_t, (0, _Q * c), (_R, _Q * (c + 1)))
        p = jax.nn.sigmoid(x)
        pt = jnp.where(t == 1.0, p, 1.0 - p)
        at = jnp.where(t == 1.0, 1000.0, 1.0)
        qf = 1.0 - pt
        w = at * qf * qf
        bce = jnp.maximum(x, 0.0) - x * t + jnp.log1p(jnp.exp(-jnp.abs(x)))
        cls_sum += jnp.sum(w * bce)

    sl1_sum = 0.0
    ce_sum = 0.0
    npos = 0.0
    rows7 = jax.lax.broadcasted_iota(jnp.int32, (7, _SUB, _LANE), 0)
    for a in range(2):
        mask = (rt_ref[0, 9 * a] == 1.0).astype(jnp.float32)
        npos += jnp.sum(mask)
        s = rg_ref[0, 9 * a:9 * a + 7]
        if a == 0:
            s = jnp.where(rows7 == 6, jnp.tanh(s), s)
        d = s - rt_ref[0, 9 * a + 1:9 * a + 8]
        ad = jnp.abs(d)
        sl1 = jnp.where(ad < 1.0, 0.5 * d * d, ad - 0.5)
        sl1_sum += jnp.sum(sl1 * mask[None])
        z0 = rg_ref[0, 9 * a + 7]
        z1 = rg_ref[0, 9 * a + 8]
        tc = rt_ref[0, 9 * a + 8]
        diff = jnp.where(tc == 1.0, z0 - z1, z1 - z0)
        ce = jnp.maximum(diff, 0.0) + jnp.log1p(jnp.exp(-jnp.abs(diff)))
        ce_sum += jnp.sum(ce * mask)

    acc_ref[0] += cls_sum
    acc_ref[1] += sl1_sum
    acc_ref[2] += ce_sum
    acc_ref[3] += npos

    @pl.when(b == _B - 1)
    def _final():
        n_pos = acc_ref[3]
        cls_loss = acc_ref[0] / _CLS_TOTAL
        reg_loss = acc_ref[1] / (n_pos * 7.0)
        ort_loss = acc_ref[2] / n_pos
        loss = B_CLS * cls_loss + B_ORT * ort_loss + B_REG * reg_loss
        out_ref[...] = jnp.full((1, 1), loss, dtype=jnp.float32)


def kernel(cls_tensor, reg_tensor, cls_targets, reg_targets):
    x = cls_tensor.reshape(_B, 2, _R, _Q)
    t = cls_targets.reshape(_B, _R, 128)
    rg = reg_tensor.reshape(_B, 18, _SUB, _LANE)
    rt = (reg_targets.reshape(_B, _P, 2, 9)
          .transpose(0, 2, 3, 1)
          .reshape(_B, 18, _SUB, _LANE))
    s_t = _build_selector()

    out = pl.pallas_call(
        _loss_kernel,
        grid=(_B,),
        in_specs=[
            pl.BlockSpec((1, 2, _R, _Q), lambda b: (b, 0, 0, 0)),
            pl.BlockSpec((1, _R, 128), lambda b: (b, 0, 0)),
            pl.BlockSpec((1, 18, _SUB, _LANE), lambda b: (b, 0, 0, 0)),
            pl.BlockSpec((1, 18, _SUB, _LANE), lambda b: (b, 0, 0, 0)),
            pl.BlockSpec((128, 128), lambda b: (0, 0)),
        ],
        out_specs=pl.BlockSpec((1, 1), lambda b: (0, 0)),
        out_shape=jax.ShapeDtypeStruct((1, 1), jnp.float32),
        scratch_shapes=[pltpu.SMEM((4,), jnp.float32)],
    )(x, t, rg, rt, s_t)
    return out[0, 0]
